# Initial kernel scaffold; baseline (speedup 1.0000x reference)
import jax
import jax.numpy as jnp
from jax.experimental import pallas as pl


def kernel(lS_o, lS_i, tables):
    raise NotImplementedError("write your pallas kernel here")


# speedup vs baseline: 2.4408x; 2.4408x over previous
import functools

import jax
import jax.numpy as jnp
from jax import lax
from jax.experimental import pallas as pl
from jax.experimental.pallas import tpu as pltpu
from jax.experimental.pallas import tpu_sc as plsc

_NUM_TABLES = 26
_VOCAB = 100000
_DIM = 16
_NUM_IDX = 16384
_NC = 2
_NS = 16
_TPC = _NUM_TABLES // _NC
_CHUNK = _NUM_IDX // _NS
_UNROLL = 4


def _sc_body(tabs_hbm, lsi_hbm, out_hbm, idx_buf, rows_buf, partials,
             idxsc, shared, sem0, sem1):
    cid = lax.axis_index("c")
    sid = lax.axis_index("s")
    sems = (sem0, sem1)

    zeros = jnp.zeros((_DIM,), jnp.float32)
    for r in range(_NS):
        partials[r, :] = zeros
    idxsc[...] = lax.iota(jnp.int32, _NS)

    @pl.when(sid == 0)
    def _():
        pltpu.sync_copy(partials, shared)

    plsc.subcore_barrier()

    def _load_and_fire(t, buf):
        k = cid * _TPC + t
        pltpu.sync_copy(lsi_hbm.at[k, pl.ds(sid * _CHUNK, _CHUNK)],
                        idx_buf.at[buf])
        kv = k * _VOCAB

        def _bias(j, _):
            v = idx_buf[buf, pl.ds(j * 16, 16)]
            idx_buf[buf, pl.ds(j * 16, 16)] = v + kv
            return 0

        lax.fori_loop(0, _CHUNK // 16, _bias, 0)
        return pltpu.async_copy(tabs_hbm.at[idx_buf.at[buf]],
                                rows_buf.at[buf], sems[buf])

    handles = [None, None]
    handles[0] = _load_and_fire(0, 0)
    for t in range(_TPC):
        buf = t & 1
        if t + 1 < _TPC:
            handles[1 - buf] = _load_and_fire(t + 1, 1 - buf)
        handles[buf].wait()

        def _acc(i, carry):
            return tuple(
                carry[u] + rows_buf[buf, i * _UNROLL + u, :]
                for u in range(_UNROLL)
            )

        accs = lax.fori_loop(0, _CHUNK // _UNROLL, _acc,
                             tuple(zeros for _ in range(_UNROLL)))
        partials[t, :] = (accs[0] + accs[1]) + (accs[2] + accs[3])

    pltpu.sync_copy(partials, shared.at[idxsc], add=True)
    plsc.subcore_barrier()

    @pl.when(sid == 0)
    def _():
        pltpu.sync_copy(shared, partials)
        pltpu.sync_copy(partials.at[pl.ds(0, _TPC)], out_hbm.at[cid])


_sc_lookup = functools.partial(
    pl.kernel,
    mesh=plsc.VectorSubcoreMesh(core_axis_name="c", subcore_axis_name="s"),
    out_type=jax.ShapeDtypeStruct((_NC, _TPC, _DIM), jnp.float32),
    compiler_params=pltpu.CompilerParams(use_tc_tiling_on_sc=False),
    scratch_types=[
        pltpu.VMEM((2, _CHUNK), jnp.int32),
        pltpu.VMEM((2, _CHUNK, _DIM), jnp.float32),
        pltpu.VMEM((_NS, _DIM), jnp.float32),
        pltpu.VMEM((_NS,), jnp.int32),
        pltpu.VMEM_SHARED((_NS, _DIM), jnp.float32),
        pltpu.SemaphoreType.DMA,
        pltpu.SemaphoreType.DMA,
    ],
)(_sc_body)


@jax.jit
def _run(lS_i, tables):
    flat = tables.reshape(_NUM_TABLES * _VOCAB, _DIM)
    return _sc_lookup(flat, lS_i).reshape(_NUM_TABLES, _DIM)


def kernel(lS_o, lS_i, tables):
    out = _run(lS_i, tables)
    return tuple(out[k:k + 1] for k in range(_NUM_TABLES))


# speedup vs baseline: 9.4714x; 3.8804x over previous
import functools

import jax
import jax.numpy as jnp
from jax import lax
from jax.experimental import pallas as pl
from jax.experimental.pallas import tpu as pltpu
from jax.experimental.pallas import tpu_sc as plsc

_NUM_TABLES = 26
_VOCAB = 100000
_DIM = 16
_NUM_IDX = 16384
_NC = 2
_NS = 16
_BINS = 102400
_VBLK = 12800
_NBLK = _BINS // _VBLK


def _sc_hist_body(lsi_hbm, out_hbm, idx_v, counts_v, sem):
    k = lax.axis_index("s") * _NC + lax.axis_index("c")

    @pl.when(k < _NUM_TABLES)
    def _():
        h = pltpu.async_copy(lsi_hbm.at[k], idx_v, sem)

        zeros = jnp.zeros((16,), jnp.float32)

        def _zero(j, _):
            counts_v[pl.ds(j * 16, 16)] = zeros
            return 0

        lax.fori_loop(0, _BINS // 16, _zero, 0)
        h.wait()

        ones = jnp.ones((16,), jnp.float32)

        def _hist(j, _):
            for u in range(4):
                jj = j * 4 + u
                v = idx_v[jj >> 3, pl.ds((jj & 7) * 16, 16)]
                cnt, last = plsc.scan_count(v)
                plsc.addupdate_scatter(
                    counts_v, [v], cnt.astype(jnp.float32) * ones, mask=last)
            return 0

        lax.fori_loop(0, _NUM_IDX // 16 // 4, _hist, 0)
        pltpu.sync_copy(counts_v, out_hbm.at[pl.ds(k * _BINS, _BINS)])


_sc_hist = functools.partial(
    pl.kernel,
    mesh=plsc.VectorSubcoreMesh(core_axis_name="c", subcore_axis_name="s"),
    out_type=jax.ShapeDtypeStruct((_NUM_TABLES * _BINS,), jnp.float32),
    compiler_params=pltpu.CompilerParams(
        use_tc_tiling_on_sc=False, needs_layout_passes=False),
    scratch_types=[
        pltpu.VMEM((128, 128), jnp.int32),
        pltpu.VMEM((_BINS,), jnp.float32),
        pltpu.SemaphoreType.DMA,
    ],
)(_sc_hist_body)


def _tc_body(cnt_ref, tt_ref, out_ref):
    j = pl.program_id(1)
    tb = tt_ref[0]
    cb = cnt_ref[0, pl.ds(j, 1), :]
    col = j * _VBLK + lax.broadcasted_iota(jnp.int32, (1, _VBLK), 1)
    prod = jnp.where(col < _VOCAB, tb * cb, 0.0)
    partial = jnp.sum(prod, axis=1)

    @pl.when(j == 0)
    def _():
        out_ref[...] = jnp.zeros_like(out_ref)

    out_ref[0, 0, :] += partial


_tc_reduce = pl.pallas_call(
    _tc_body,
    grid=(_NUM_TABLES, _NBLK),
    in_specs=[
        pl.BlockSpec((1, _NBLK, _VBLK), lambda k, j: (k, 0, 0)),
        pl.BlockSpec((1, _DIM, _VBLK), lambda k, j: (k, 0, j)),
    ],
    out_specs=pl.BlockSpec((1, 1, _DIM), lambda k, j: (k, 0, 0)),
    out_shape=jax.ShapeDtypeStruct((_NUM_TABLES, 1, _DIM), jnp.float32),
    compiler_params=pltpu.CompilerParams(
        dimension_semantics=("parallel", "arbitrary")),
)


@jax.jit
def _run(lS_i, tables):
    tt = jnp.swapaxes(tables, 1, 2)
    lsi3 = lS_i.reshape(_NUM_TABLES, 128, 128)
    counts = _sc_hist(lsi3)
    cnt3 = counts.reshape(_NUM_TABLES, _NBLK, _VBLK)
    out = _tc_reduce(cnt3, tt)
    return out.reshape(_NUM_TABLES, _DIM)


def kernel(lS_o, lS_i, tables):
    out = _run(lS_i, tables)
    return tuple(out[k:k + 1] for k in range(_NUM_TABLES))


# speedup vs baseline: 17.1966x; 1.8156x over previous
import functools

import jax
import jax.numpy as jnp
from jax import lax
from jax.experimental import pallas as pl
from jax.experimental.pallas import tpu as pltpu
from jax.experimental.pallas import tpu_sc as plsc

_NUM_TABLES = 26
_VOCAB = 100000
_DIM = 16
_NUM_IDX = 16384
_NC = 2
_NS = 16
_BINS = 102400
_VBLK = 12800
_NBLK = _BINS // _VBLK


def _sc_hist_body(lsi_hbm, out_hbm, idx_v, counts_v, sem):
    k = lax.axis_index("s") * _NC + lax.axis_index("c")

    @pl.when(k < _NUM_TABLES)
    def _():
        h = pltpu.async_copy(lsi_hbm.at[k], idx_v, sem)

        zeros = jnp.zeros((16,), jnp.float32)

        def _zero(j, _):
            counts_v[pl.ds(j * 16, 16)] = zeros
            return 0

        lax.fori_loop(0, _BINS // 16, _zero, 0)
        h.wait()

        ones = jnp.ones((16,), jnp.float32)

        def _hist(j, _):
            for u in range(4):
                jj = j * 4 + u
                v = idx_v[jj >> 3, pl.ds((jj & 7) * 16, 16)]
                cnt, last = plsc.scan_count(v)
                plsc.addupdate_scatter(
                    counts_v, [v], cnt.astype(jnp.float32) * ones, mask=last)
            return 0

        lax.fori_loop(0, _NUM_IDX // 16 // 4, _hist, 0)
        pltpu.sync_copy(counts_v, out_hbm.at[pl.ds(k * _BINS, _BINS)])


_sc_hist = functools.partial(
    pl.kernel,
    mesh=plsc.VectorSubcoreMesh(core_axis_name="c", subcore_axis_name="s"),
    out_type=jax.ShapeDtypeStruct((_NUM_TABLES * _BINS,), jnp.float32),
    compiler_params=pltpu.CompilerParams(
        use_tc_tiling_on_sc=False, needs_layout_passes=False),
    scratch_types=[
        pltpu.VMEM((128, 128), jnp.int32),
        pltpu.VMEM((_BINS,), jnp.float32),
        pltpu.SemaphoreType.DMA,
    ],
)(_sc_hist_body)


def _tc_body(cnt_ref, tt_ref, out_ref):
    tb = tt_ref[0]
    acc = jnp.zeros((_DIM,), jnp.float32)
    for r in range(_NBLK):
        lo = r * _VBLK
        hi = min((r + 1) * _VBLK, _VOCAB)
        cbr = cnt_ref[0, pl.ds(r, 1), pl.ds(0, hi - lo)]
        acc += jnp.sum(tb[:, lo:hi] * cbr, axis=1)
    out_ref[0, 0, :] = acc


_tc_reduce = pl.pallas_call(
    _tc_body,
    grid=(_NUM_TABLES,),
    in_specs=[
        pl.BlockSpec((1, _NBLK, _VBLK), lambda k: (k, 0, 0)),
        pl.BlockSpec((1, _DIM, _VOCAB), lambda k: (k, 0, 0)),
    ],
    out_specs=pl.BlockSpec((1, 1, _DIM), lambda k: (k, 0, 0)),
    out_shape=jax.ShapeDtypeStruct((_NUM_TABLES, 1, _DIM), jnp.float32),
    compiler_params=pltpu.CompilerParams(
        dimension_semantics=("arbitrary",)),
)


@jax.jit
def _run(lS_i, tables):
    tt = jnp.swapaxes(tables, 1, 2)
    lsi3 = lS_i.reshape(_NUM_TABLES, 128, 128)
    counts = _sc_hist(lsi3)
    cnt3 = counts.reshape(_NUM_TABLES, _NBLK, _VBLK)
    out = _tc_reduce(cnt3, tt)
    return out.reshape(_NUM_TABLES, _DIM)


def kernel(lS_o, lS_i, tables):
    out = _run(lS_i, tables)
    return tuple(out[k:k + 1] for k in range(_NUM_TABLES))


# speedup vs baseline: 22.8565x; 1.3291x over previous
import functools

import jax
import jax.numpy as jnp
from jax import lax
from jax.experimental import pallas as pl
from jax.experimental.pallas import tpu as pltpu
from jax.experimental.pallas import tpu_sc as plsc

_NUM_TABLES = 26
_VOCAB = 100000
_DIM = 16
_NUM_IDX = 16384
_NC = 2
_NS = 16
_BINS = 102400
_VBLK = 12800
_NBLK = _BINS // _VBLK


def _sc_hist_body(lsi_hbm, out_hbm, idx_v, counts_v, sem):
    k = lax.axis_index("s") * _NC + lax.axis_index("c")

    @pl.when(k < _NUM_TABLES)
    def _():
        h = pltpu.async_copy(lsi_hbm.at[k], idx_v, sem)

        zeros = jnp.zeros((16,), jnp.float32)

        def _zero(j, _):
            for u in range(8):
                counts_v[pl.ds((j * 8 + u) * 16, 16)] = zeros
            return 0

        lax.fori_loop(0, _BINS // 16 // 8, _zero, 0)
        h.wait()

        def _hist(j, _):
            vs = [idx_v[(j * 8 + u) >> 3, pl.ds(((j * 8 + u) & 7) * 16, 16)]
                  for u in range(8)]
            cls = [plsc.scan_count(v) for v in vs]
            for v, (cnt, last) in zip(vs, cls):
                plsc.addupdate_scatter(
                    counts_v, [v], cnt.astype(jnp.float32), mask=last)
            return 0

        lax.fori_loop(0, _NUM_IDX // 16 // 8, _hist, 0)
        pltpu.sync_copy(counts_v, out_hbm.at[pl.ds(k * _BINS, _BINS)])


_sc_hist = functools.partial(
    pl.kernel,
    mesh=plsc.VectorSubcoreMesh(core_axis_name="c", subcore_axis_name="s"),
    out_type=jax.ShapeDtypeStruct((_NUM_TABLES * _BINS,), jnp.float32),
    compiler_params=pltpu.CompilerParams(
        use_tc_tiling_on_sc=False, needs_layout_passes=False),
    scratch_types=[
        pltpu.VMEM((128, 128), jnp.int32),
        pltpu.VMEM((_BINS,), jnp.float32),
        pltpu.SemaphoreType.DMA,
    ],
)(_sc_hist_body)


def _tc_body(cnt_ref, tt_ref, out_ref):
    tb = tt_ref[0]
    acc = jnp.zeros((_DIM,), jnp.float32)
    for r in range(_NBLK):
        lo = r * _VBLK
        hi = min((r + 1) * _VBLK, _VOCAB)
        cbr = cnt_ref[0, pl.ds(r, 1), pl.ds(0, hi - lo)]
        acc += jnp.sum(tb[:, lo:hi] * cbr, axis=1)
    out_ref[0, 0, :] = acc


_tc_reduce = pl.pallas_call(
    _tc_body,
    grid=(_NUM_TABLES,),
    in_specs=[
        pl.BlockSpec((1, _NBLK, _VBLK), lambda k: (k, 0, 0)),
        pl.BlockSpec((1, _DIM, _VOCAB), lambda k: (k, 0, 0)),
    ],
    out_specs=pl.BlockSpec((1, 1, _DIM), lambda k: (k, 0, 0)),
    out_shape=jax.ShapeDtypeStruct((_NUM_TABLES, 1, _DIM), jnp.float32),
    compiler_params=pltpu.CompilerParams(
        dimension_semantics=("arbitrary",)),
)


@jax.jit
def _run(lS_i, tables):
    tt = jnp.swapaxes(tables, 1, 2)
    lsi3 = lS_i.reshape(_NUM_TABLES, 128, 128)
    counts = _sc_hist(lsi3)
    cnt3 = counts.reshape(_NUM_TABLES, _NBLK, _VBLK)
    out = _tc_reduce(cnt3, tt)
    return out.reshape(_NUM_TABLES, _DIM)


def kernel(lS_o, lS_i, tables):
    out = _run(lS_i, tables)
    return tuple(out[k:k + 1] for k in range(_NUM_TABLES))
